# baseline (speedup 1.0000x reference)
import functools

import jax
import jax.numpy as jnp
from jax import lax
from jax.experimental import pallas as pl
from jax.experimental.pallas import tpu as pltpu
from jax.experimental.pallas import tpu_sc as plsc

_DIM = 32
_BATCH = 16384
_FIELDS = 26
_BB = 512
_NBUF = 3


@functools.lru_cache(maxsize=None)
def _build(batch, fields, dim):
    info = plsc.get_sparse_core_info()
    nc, ns = info.num_cores, info.num_subcores
    nw = nc * ns
    nbb = batch // _BB
    n_pairs = fields * nbb
    per_w = n_pairs // nw
    nbuf = _NBUF
    assert n_pairs % nw == 0 and per_w >= nbuf

    mesh = plsc.VectorSubcoreMesh(core_axis_name="c", subcore_axis_name="s")

    @functools.partial(
        pl.kernel,
        mesh=mesh,
        out_type=jax.ShapeDtypeStruct((fields, batch, dim), jnp.float32),
        scratch_types=[pltpu.VMEM((_BB,), jnp.int32)] * per_w
        + [pltpu.VMEM((_BB, dim), jnp.float32)] * nbuf
        + [pltpu.SemaphoreType.DMA] * (1 + 2 * nbuf),
        compiler_params=pltpu.CompilerParams(use_tc_tiling_on_sc=False),
    )
    def gather_kernel(xt_hbm, table_hbm, out_hbm, *rest):
        ibuf = rest[:per_w]
        gbuf = rest[per_w : per_w + nbuf]
        isem = rest[per_w + nbuf]
        gsem = rest[per_w + nbuf + 1 : per_w + 2 * nbuf + 1]
        ssem = rest[per_w + 2 * nbuf + 1 :]
        wid = lax.axis_index("s") * nc + lax.axis_index("c")
        t0 = wid * per_w

        def fld(k):
            return (t0 + k) // nbb

        def bb(k):
            return ((t0 + k) % nbb) * _BB

        def icopy(k):
            return pltpu.make_async_copy(
                xt_hbm.at[fld(k), pl.ds(bb(k), _BB)], ibuf[k], isem
            )

        def gcopy(k, b):
            return pltpu.make_async_copy(
                table_hbm.at[ibuf[k]], gbuf[b], gsem[b]
            )

        def scopy(k, b):
            return pltpu.make_async_copy(
                gbuf[b],
                out_hbm.at[fld(k), pl.ds(bb(k), _BB), :],
                ssem[b],
            )

        for k in range(per_w):
            icopy(k).start()
        for k in range(per_w):
            icopy(k).wait()

        for b in range(nbuf):
            gcopy(b, b).start()
        for k in range(per_w):
            b = k % nbuf
            gcopy(k, b).wait()
            scopy(k, b).start()
            nxt = k + nbuf
            if nxt < per_w:
                scopy(k, b).wait()
                gcopy(nxt, b).start()
        for k in range(per_w - min(nbuf, per_w), per_w):
            scopy(k, k % nbuf).wait()

    return gather_kernel


@functools.lru_cache(maxsize=None)
def _build_k1(vocab, dim):
    info = plsc.get_sparse_core_info()
    nc, ns = info.num_cores, info.num_subcores
    nw = nc * ns
    rows = vocab * dim // 128
    main_cols = (vocab // 128) * 128 - ((vocab // 128) % nw) * 128
    n_blocks = main_cols // 128
    per_w = n_blocks // nw
    n_extra = vocab // 128 - n_blocks
    tail = vocab - (vocab // 128) * 128
    assert per_w % 2 == 0 and per_w >= 6

    mesh = plsc.VectorSubcoreMesh(core_axis_name="c", subcore_axis_name="s")

    @functools.partial(
        pl.kernel,
        mesh=mesh,
        out_type=jax.ShapeDtypeStruct((rows, 128), jnp.float32),
        scratch_types=[pltpu.VMEM((dim, 128), jnp.float32)] * 2
        + [pltpu.VMEM((dim, 128), jnp.float32)] * 2
        + [pltpu.VMEM((dim, tail), jnp.float32), pltpu.VMEM((tail * dim // 128, 128), jnp.float32)]
        + [pltpu.SemaphoreType.DMA] * 4,
        compiler_params=pltpu.CompilerParams(needs_layout_passes=False),
    )
    def t_kernel(wt_hbm, out_hbm, *rest):
        ibuf = rest[0:2]
        obuf = rest[2:4]
        ibuf_t, obuf_t = rest[4], rest[5]
        isem = rest[6:8]
        osem = rest[8:10]
        wid = lax.axis_index("s") * nc + lax.axis_index("c")
        lanes = lax.iota(jnp.int32, 16)

        def c0(m):
            return pl.multiple_of((wid * per_w + m) * 128, 128)

        def icopy(m, b):
            return pltpu.make_async_copy(
                wt_hbm.at[:, pl.ds(c0(m), 128)], ibuf[b], isem[b]
            )

        def ocopy(m, b):
            return pltpu.make_async_copy(
                obuf[b],
                out_hbm.at[pl.ds(pl.multiple_of(c0(m) // 4, 32), dim), :],
                osem[b],
            )

        def transpose(src, dst, njr):
            @plsc.parallel_loop(0, njr, unroll=4)
            def body(jr):
                base = jnp.full((16,), 0, jnp.int32) + jr * 4
                vlocs = [base + q for q in range(4)]
                for t in range(8):
                    cvec = (t % 2) * 16 + lanes
                    v = plsc.load_gather(src, [cvec, vlocs[t // 2]])
                    dst[jr, pl.ds(16 * t, 16)] = v

        def step(m, b, with_osem, more_icopy):
            icopy(0, b).wait()
            if with_osem:
                ocopy(0, b).wait()
            transpose(ibuf[b], obuf[b], dim)
            ocopy(m, b).start()
            if more_icopy:
                icopy(m + 2, b).start()

        icopy(0, 0).start()
        icopy(1, 1).start()
        step(0, 0, False, True)
        step(1, 1, False, True)

        def group(g, carry):
            m = 2 * g + 2
            step(m, 0, True, True)
            step(m + 1, 1, True, True)
            return carry

        lax.fori_loop(0, (per_w - 4) // 2, group, 0)

        step(per_w - 2, 0, True, False)
        step(per_w - 1, 1, True, False)
        ocopy(0, 0).wait()
        ocopy(0, 1).wait()

        @pl.when(wid < n_extra)
        def _():
            ce = pl.multiple_of((n_blocks + wid) * 128, 128)
            cp = pltpu.make_async_copy(
                wt_hbm.at[:, pl.ds(ce, 128)], ibuf[0], isem[0]
            )
            cp.start()
            cp.wait()
            transpose(ibuf[0], obuf[0], dim)
            cp2 = pltpu.make_async_copy(
                obuf[0],
                out_hbm.at[pl.ds(pl.multiple_of(ce // 4, 32), dim), :],
                osem[0],
            )
            cp2.start()
            cp2.wait()

        @pl.when(wid == n_extra)
        def _():
            ct = (n_blocks + n_extra) * 128
            cp = pltpu.make_async_copy(
                wt_hbm.at[:, pl.ds(ct, tail)], ibuf_t, isem[0]
            )
            cp.start()
            cp.wait()
            transpose(ibuf_t, obuf_t, tail * dim // 128)
            cp2 = pltpu.make_async_copy(
                obuf_t,
                out_hbm.at[pl.ds(ct // 4, tail * dim // 128), :],
                osem[0],
            )
            cp2.start()
            cp2.wait()

    return t_kernel


def kernel(x, weight):
    xt = jnp.swapaxes(x, 0, 1).astype(jnp.int32)
    table = _build_k1(weight.shape[0], weight.shape[1])(weight.T)
    table = table.reshape(weight.shape)
    out_t = _build(_BATCH, _FIELDS, _DIM)(xt, table)
    return jnp.swapaxes(out_t, 0, 1)


# speedup vs baseline: 1.5508x; 1.5508x over previous
import functools

import jax
import jax.numpy as jnp
from jax import lax
from jax.experimental import pallas as pl
from jax.experimental.pallas import tpu as pltpu
from jax.experimental.pallas import tpu_sc as plsc

_DIM = 32
_BATCH = 16384
_FIELDS = 26
_BB = 512
_NBUF = 3


@functools.lru_cache(maxsize=None)
def _build(batch, fields, dim):
    info = plsc.get_sparse_core_info()
    nc, ns = info.num_cores, info.num_subcores
    nw = nc * ns
    nbb = batch // _BB
    n_pairs = fields * nbb
    per_w = n_pairs // nw
    nbuf = _NBUF
    assert n_pairs % nw == 0 and per_w >= nbuf

    mesh = plsc.VectorSubcoreMesh(core_axis_name="c", subcore_axis_name="s")

    @functools.partial(
        pl.kernel,
        mesh=mesh,
        out_type=jax.ShapeDtypeStruct((fields, batch, dim), jnp.float32),
        scratch_types=[pltpu.VMEM((_BB,), jnp.int32)] * per_w
        + [pltpu.VMEM((_BB, dim), jnp.float32)] * nbuf
        + [pltpu.SemaphoreType.DMA] * (1 + 2 * nbuf),
        compiler_params=pltpu.CompilerParams(use_tc_tiling_on_sc=False),
    )
    def gather_kernel(xt_hbm, table_hbm, out_hbm, *rest):
        ibuf = rest[:per_w]
        gbuf = rest[per_w : per_w + nbuf]
        isem = rest[per_w + nbuf]
        gsem = rest[per_w + nbuf + 1 : per_w + 2 * nbuf + 1]
        ssem = rest[per_w + 2 * nbuf + 1 :]
        wid = lax.axis_index("s") * nc + lax.axis_index("c")
        t0 = wid * per_w

        def fld(k):
            return (t0 + k) // nbb

        def bb(k):
            return ((t0 + k) % nbb) * _BB

        def icopy(k):
            return pltpu.make_async_copy(
                xt_hbm.at[fld(k), pl.ds(bb(k), _BB)], ibuf[k], isem
            )

        def gcopy(k, b):
            return pltpu.make_async_copy(
                table_hbm.at[ibuf[k]], gbuf[b], gsem[b]
            )

        def scopy(k, b):
            return pltpu.make_async_copy(
                gbuf[b],
                out_hbm.at[fld(k), pl.ds(bb(k), _BB), :],
                ssem[b],
            )

        for k in range(per_w):
            icopy(k).start()
        for k in range(per_w):
            icopy(k).wait()

        for b in range(nbuf):
            gcopy(b, b).start()
        for k in range(per_w):
            b = k % nbuf
            gcopy(k, b).wait()
            scopy(k, b).start()
            nxt = k + nbuf
            if nxt < per_w:
                scopy(k, b).wait()
                gcopy(nxt, b).start()
        for k in range(per_w - min(nbuf, per_w), per_w):
            scopy(k, k % nbuf).wait()

    return gather_kernel


@functools.lru_cache(maxsize=None)
def _build_k1(vocab, dim):
    info = plsc.get_sparse_core_info()
    nc, ns = info.num_cores, info.num_subcores
    nw = nc * ns
    rows = vocab * dim // 128
    main_cols = (vocab // 128) * 128 - ((vocab // 128) % nw) * 128
    n_blocks = main_cols // 128
    per_w = n_blocks // nw
    n_extra = vocab // 128 - n_blocks
    tail = vocab - (vocab // 128) * 128
    assert per_w % 2 == 0 and per_w >= 6

    mesh = plsc.VectorSubcoreMesh(core_axis_name="c", subcore_axis_name="s")

    @functools.partial(
        pl.kernel,
        mesh=mesh,
        out_type=jax.ShapeDtypeStruct((rows * 128,), jnp.float32),
        scratch_types=[pltpu.VMEM((dim, 128), jnp.float32)] * 2
        + [pltpu.VMEM((128 * dim,), jnp.float32)] * 2
        + [pltpu.VMEM((dim, tail), jnp.float32), pltpu.VMEM((tail * dim,), jnp.float32)]
        + [pltpu.SemaphoreType.DMA] * 4,
        compiler_params=pltpu.CompilerParams(needs_layout_passes=False),
    )
    def t_kernel(wt_hbm, out_hbm, *rest):
        ibuf = rest[0:2]
        obuf = rest[2:4]
        ibuf_t, obuf_t = rest[4], rest[5]
        isem = rest[6:8]
        osem = rest[8:10]
        wid = lax.axis_index("s") * nc + lax.axis_index("c")
        lanes = lax.iota(jnp.int32, 16)

        def c0(m):
            return pl.multiple_of((wid * per_w + m) * 128, 128)

        def icopy(m, b):
            return pltpu.make_async_copy(
                wt_hbm.at[:, pl.ds(c0(m), 128)], ibuf[b], isem[b]
            )

        def ocopy(m, b):
            return pltpu.make_async_copy(
                obuf[b],
                out_hbm.at[pl.ds(pl.multiple_of(c0(m) * dim, 128 * dim), 128 * dim)],
                osem[b],
            )

        rots = [(lanes + k) & 15 for k in range(16)]
        rot32 = [rots[k] * dim + lanes for k in range(16)]

        def transpose(src, dst_flat, ncols):
            @plsc.parallel_loop(0, ncols // 16, unroll=2)
            def body(u):
                vbase = u * 16
                for k in range(16):
                    vv = vbase + rots[k]
                    for h in (0, 16):
                        v = plsc.load_gather(src, [lanes + h, vv])
                        plsc.store_scatter(
                            dst_flat, [vbase * dim + h + rot32[k]], v
                        )

        def step(m, b, with_osem, more_icopy):
            icopy(0, b).wait()
            if with_osem:
                ocopy(0, b).wait()
            transpose(ibuf[b], obuf[b], 128)
            ocopy(m, b).start()
            if more_icopy:
                icopy(m + 2, b).start()

        icopy(0, 0).start()
        icopy(1, 1).start()
        step(0, 0, False, True)
        step(1, 1, False, True)

        def group(g, carry):
            m = 2 * g + 2
            step(m, 0, True, True)
            step(m + 1, 1, True, True)
            return carry

        lax.fori_loop(0, (per_w - 4) // 2, group, 0)

        step(per_w - 2, 0, True, False)
        step(per_w - 1, 1, True, False)
        ocopy(0, 0).wait()
        ocopy(0, 1).wait()

        @pl.when(wid < n_extra)
        def _():
            ce = pl.multiple_of((n_blocks + wid) * 128, 128)
            cp = pltpu.make_async_copy(
                wt_hbm.at[:, pl.ds(ce, 128)], ibuf[0], isem[0]
            )
            cp.start()
            cp.wait()
            transpose(ibuf[0], obuf[0], 128)
            cp2 = pltpu.make_async_copy(
                obuf[0],
                out_hbm.at[pl.ds(pl.multiple_of(ce * dim, 128 * dim), 128 * dim)],
                osem[0],
            )
            cp2.start()
            cp2.wait()

        @pl.when(wid == n_extra)
        def _():
            ct = (n_blocks + n_extra) * 128
            cp = pltpu.make_async_copy(
                wt_hbm.at[:, pl.ds(ct, tail)], ibuf_t, isem[0]
            )
            cp.start()
            cp.wait()
            transpose(ibuf_t, obuf_t, tail)
            cp2 = pltpu.make_async_copy(
                obuf_t,
                out_hbm.at[pl.ds(ct * dim, tail * dim)],
                osem[0],
            )
            cp2.start()
            cp2.wait()

    return t_kernel


def kernel(x, weight):
    xt = jnp.swapaxes(x, 0, 1).astype(jnp.int32)
    table = _build_k1(weight.shape[0], weight.shape[1])(weight.T)
    table = table.reshape(weight.shape)
    out_t = _build(_BATCH, _FIELDS, _DIM)(xt, table)
    return jnp.swapaxes(out_t, 0, 1)
